# baseline (speedup 1.0000x reference)
import functools

import jax
import jax.numpy as jnp
from jax import lax
from jax.experimental import pallas as pl
from jax.experimental.pallas import tpu as pltpu
from jax.experimental.pallas import tpu_sc as plsc

V = 100000
D = 128
D2 = 2 * D
DW = D
DH = D // 2
B = 16384
NNEG = 20
NC, NS = 2, 16
NW = NC * NS
BPW = B // NW
C = 16
CN = C * NNEG
NG = CN // 16
NCHUNK = BPW // C

_PC = (
    0.00022841986642679486,
    0.002200562996050353,
    -0.0049591490971503671,
    -0.079778088603572717,
    0.72134636444934774,
    -0.36651556254295736,
)


def _logsp(t):
    r = t * _PC[0] + _PC[1]
    for c in _PC[2:]:
        r = r * t + c
    return r


_TX_ROWS = 1000


def _bf16_code(x):
    code = lax.bitcast_convert_type(x.astype(jnp.bfloat16), jnp.uint16)
    return code.astype(jnp.uint32)


def _pack_half(h):
    lo = _bf16_code(h[:, :DH])
    hi = _bf16_code(h[:, DH:])
    return lax.bitcast_convert_type(lo | (hi << 16), jnp.int32)


def _tx_one(vec):
    w = vec[:, :D]
    Wc = vec[:, D:]
    z = jax.nn.sigmoid(w)
    Z = z + jax.nn.sigmoid(Wc) * (1.0 - z)
    return jnp.concatenate([_pack_half(z), _pack_half(Z)], axis=1)


def _tx_kernel(word_ref, ctx_ref, ow_ref, oc_ref):
    ow_ref[...] = _tx_one(word_ref[...])
    oc_ref[...] = _tx_one(ctx_ref[...])


def _transform_tables(W_word, W_ctx):
    in_spec = pl.BlockSpec((_TX_ROWS, D2), lambda i: (i, 0))
    out_spec = pl.BlockSpec((_TX_ROWS, DW), lambda i: (i, 0))
    return pl.pallas_call(
        _tx_kernel,
        grid=(V // _TX_ROWS,),
        in_specs=[in_spec, in_spec],
        out_specs=[out_spec, out_spec],
        out_shape=[jax.ShapeDtypeStruct((V, DW), jnp.int32)] * 2,
    )(W_word, W_ctx)


_sc_mesh = plsc.VectorSubcoreMesh(core_axis_name="c", subcore_axis_name="s")


def _unpk(word_i32):
    lo = plsc.bitcast(lax.shift_left(word_i32, 16), jnp.float32)
    hi = plsc.bitcast(word_i32, jnp.float32)
    return lo, hi


@functools.partial(
    pl.kernel,
    out_type=[
        jax.ShapeDtypeStruct((B,), jnp.float32),
        jax.ShapeDtypeStruct((B,), jnp.float32),
        jax.ShapeDtypeStruct((B * NNEG,), jnp.float32),
        jax.ShapeDtypeStruct((B,), jnp.float32),
        jax.ShapeDtypeStruct((B * NNEG,), jnp.float32),
    ],
    mesh=_sc_mesh,
    compiler_params=pltpu.CompilerParams(needs_layout_passes=False),
    scratch_types=[
        pltpu.VMEM((BPW,), jnp.int32),
        pltpu.VMEM((BPW,), jnp.int32),
        pltpu.VMEM((BPW * NNEG,), jnp.int32),
        pltpu.VMEM((C, DW), jnp.int32),
        pltpu.VMEM((C, DW), jnp.int32),
        pltpu.VMEM((C, DW), jnp.int32),
        pltpu.VMEM((C, DW), jnp.int32),
        pltpu.VMEM((CN, DW), jnp.int32),
        pltpu.VMEM((CN, DW), jnp.int32),
        pltpu.VMEM((BPW,), jnp.float32),
        pltpu.VMEM((BPW,), jnp.float32),
        pltpu.VMEM((BPW * NNEG,), jnp.float32),
        pltpu.VMEM((BPW,), jnp.float32),
        pltpu.VMEM((BPW * NNEG,), jnp.float32),
        pltpu.SemaphoreType.DMA,
        pltpu.SemaphoreType.DMA,
    ],
)
def _sc_volumes(pos_u_h, pos_w_h, negf_h, zzw_h, zzc_h,
                tv_h, pv_h, nv_h, pi_h, ni_h,
                idx_u, idx_w, idx_n, rows_u0, rows_u1,
                rows_w0, rows_w1, rows_n0, rows_n1,
                o_tv, o_pv, o_nv, o_pi, o_ni, sem0, sem1):
    wid = lax.axis_index("c") * NS + lax.axis_index("s")
    base = wid * BPW

    pltpu.sync_copy(pos_u_h.at[pl.ds(base, BPW)], idx_u)
    pltpu.sync_copy(pos_w_h.at[pl.ds(base, BPW)], idx_w)
    pltpu.sync_copy(negf_h.at[pl.ds(base * NNEG, BPW * NNEG)], idx_n)

    sems = (sem0, sem1)
    rows_u = (rows_u0, rows_u1)
    rows_w = (rows_w0, rows_w1)
    rows_n = (rows_n0, rows_n1)
    zero16 = jnp.zeros((16,), jnp.float32)
    lanes = lax.iota(jnp.int32, 16)

    def issue(ci, sl):
        sem = sems[sl]
        off = pl.multiple_of(ci * C, 8)
        noff = pl.multiple_of(ci * CN, 8)
        pltpu.async_copy(zzw_h.at[idx_u.at[pl.ds(off, C)]],
                         rows_u[sl], sem)
        pltpu.async_copy(zzc_h.at[idx_w.at[pl.ds(off, C)]],
                         rows_w[sl], sem)
        pltpu.async_copy(zzc_h.at[idx_n.at[pl.ds(noff, 128)]],
                         rows_n[sl].at[pl.ds(0, 128)], sem)
        pltpu.async_copy(zzc_h.at[idx_n.at[pl.ds(noff + 128, 128)]],
                         rows_n[sl].at[pl.ds(128, 128)], sem)
        pltpu.async_copy(zzc_h.at[idx_n.at[pl.ds(noff + 256, CN - 256)]],
                         rows_n[sl].at[pl.ds(256, CN - 256)], sem)

    def drain(sl):
        sem = sems[sl]
        pltpu.make_async_copy(zzw_h.at[pl.ds(0, C)], rows_u[sl], sem).wait()
        pltpu.make_async_copy(zzc_h.at[pl.ds(0, C)], rows_w[sl], sem).wait()
        pltpu.make_async_copy(zzc_h.at[pl.ds(0, CN)], rows_n[sl], sem).wait()

    def compute(ci, sl):
        ru = rows_u[sl]
        rw = rows_w[sl]
        rn = rows_n[sl]

        def pos_d(d, carry):
            tv, pv, pi_ = carry
            dv = jnp.bitwise_and(d + lanes, DH - 1)
            uzw = plsc.load_gather(ru, [lanes, dv])
            uZw = plsc.load_gather(ru, [lanes, dv + DH])
            wzw = plsc.load_gather(rw, [lanes, dv])
            wZw = plsc.load_gather(rw, [lanes, dv + DH])
            uzl, uzh = _unpk(uzw)
            uZl, uZh = _unpk(uZw)
            wzl, wzh = _unpk(wzw)
            wZl, wZh = _unpk(wZw)
            tv = tv + _logsp(uZl - uzl) + _logsp(uZh - uzh)
            pv = pv + _logsp(wZl - wzl) + _logsp(wZh - wzh)
            pi_ = (pi_
                   + _logsp(jnp.minimum(wZl, uZl) - jnp.maximum(wzl, uzl))
                   + _logsp(jnp.minimum(wZh, uZh) - jnp.maximum(wzh, uzh)))
            return tv, pv, pi_

        tv, pv, pi_ = lax.fori_loop(0, DH, pos_d,
                                    (zero16, zero16, zero16), unroll=8)
        eoff = pl.multiple_of(ci * C, 8)
        o_tv[pl.ds(eoff, 16)] = tv
        o_pv[pl.ds(eoff, 16)] = pv
        o_pi[pl.ds(eoff, 16)] = pi_

        def grp(g, _):
            rvec = lanes + g * 16
            evec = lax.div(rvec, jnp.int32(NNEG))

            def neg_d(d, carry):
                nv, ni = carry
                dv = jnp.bitwise_and(d + lanes, DH - 1)
                nzw = plsc.load_gather(rn, [rvec, dv])
                nZw = plsc.load_gather(rn, [rvec, dv + DH])
                uzw = plsc.load_gather(ru, [evec, dv])
                uZw = plsc.load_gather(ru, [evec, dv + DH])
                nzl, nzh = _unpk(nzw)
                nZl, nZh = _unpk(nZw)
                uzl, uzh = _unpk(uzw)
                uZl, uZh = _unpk(uZw)
                nv = nv + _logsp(nZl - nzl) + _logsp(nZh - nzh)
                ni = (ni
                      + _logsp(jnp.minimum(nZl, uZl) - jnp.maximum(nzl, uzl))
                      + _logsp(jnp.minimum(nZh, uZh) - jnp.maximum(nzh, uzh)))
                return nv, ni

            nv, ni = lax.fori_loop(0, DH, neg_d, (zero16, zero16), unroll=8)
            foff = pl.multiple_of(ci * CN + g * 16, 8)
            o_nv[pl.ds(foff, 16)] = nv
            o_ni[pl.ds(foff, 16)] = ni
            return 0

        lax.fori_loop(0, NG, grp, 0)

    issue(0, 0)

    def outer(p, _):
        for b in range(2):
            ci = p * 2 + b

            @pl.when(ci + 1 < NCHUNK)
            def _():
                issue(ci + 1, 1 - b)

            drain(b)
            compute(ci, b)
        return 0

    lax.fori_loop(0, NCHUNK // 2, outer, 0)

    pltpu.sync_copy(o_tv, tv_h.at[pl.ds(base, BPW)])
    pltpu.sync_copy(o_pv, pv_h.at[pl.ds(base, BPW)])
    pltpu.sync_copy(o_nv, nv_h.at[pl.ds(base * NNEG, BPW * NNEG)])
    pltpu.sync_copy(o_pi, pi_h.at[pl.ds(base, BPW)])
    pltpu.sync_copy(o_ni, ni_h.at[pl.ds(base * NNEG, BPW * NNEG)])


def kernel(pos_u, pos_w, neg_w, W_word, W_ctx):
    zzw, zzc = _transform_tables(W_word, W_ctx)
    neg_flat = neg_w.reshape(-1)
    tv, pv, nvf, pi, nif = _sc_volumes(pos_u, pos_w, neg_flat, zzw, zzc)
    return (tv, pv, nvf.reshape(B, NNEG), pi, nif.reshape(B, NNEG))


# speedup vs baseline: 1.2780x; 1.2780x over previous
import functools

import jax
import jax.numpy as jnp
from jax import lax
from jax.experimental import pallas as pl
from jax.experimental.pallas import tpu as pltpu
from jax.experimental.pallas import tpu_sc as plsc

V = 100000
D = 128
D2 = 2 * D
DW = D
DH = D // 2
B = 16384
NNEG = 20
NC, NS = 2, 16
NW = NC * NS
BPW = B // NW
C = 16
CN = C * NNEG
NG = CN // 16
NCHUNK = BPW // C

_PC = (
    0.00022841986642679486,
    0.002200562996050353,
    -0.0049591490971503671,
    -0.079778088603572717,
    0.72134636444934774,
    -0.36651556254295736,
)


def _logsp(t):
    r = t * _PC[0] + _PC[1]
    for c in _PC[2:]:
        r = r * t + c
    return r


_TX_ROWS = 1000


def _bf16_code(x):
    code = lax.bitcast_convert_type(x.astype(jnp.bfloat16), jnp.uint16)
    return code.astype(jnp.uint32)


def _pack_half(h):
    lo = _bf16_code(h[:, :DH])
    hi = _bf16_code(h[:, DH:])
    return lax.bitcast_convert_type(lo | (hi << 16), jnp.int32)


def _tx_one(vec):
    w = vec[:, :D]
    Wc = vec[:, D:]
    z = jax.nn.sigmoid(w)
    Z = z + jax.nn.sigmoid(Wc) * (1.0 - z)
    packed = jnp.concatenate([_pack_half(z), _pack_half(Z)], axis=1)
    vol = jnp.sum(_logsp(Z - z), axis=-1, keepdims=True)
    return packed, vol


def _tx_kernel(word_ref, ctx_ref, ow_ref, oc_ref, vw_ref, vc_ref):
    ow_ref[...], vw_ref[...] = _tx_one(word_ref[...])
    oc_ref[...], vc_ref[...] = _tx_one(ctx_ref[...])


def _transform_tables(W_word, W_ctx):
    in_spec = pl.BlockSpec((_TX_ROWS, D2), lambda i: (i, 0))
    out_spec = pl.BlockSpec((_TX_ROWS, DW), lambda i: (i, 0))
    vol_spec = pl.BlockSpec((_TX_ROWS, 1), lambda i: (i, 0))
    return pl.pallas_call(
        _tx_kernel,
        grid=(V // _TX_ROWS,),
        in_specs=[in_spec, in_spec],
        out_specs=[out_spec, out_spec, vol_spec, vol_spec],
        out_shape=[jax.ShapeDtypeStruct((V, DW), jnp.int32)] * 2
        + [jax.ShapeDtypeStruct((V, 1), jnp.float32)] * 2,
    )(W_word, W_ctx)


_sc_mesh = plsc.VectorSubcoreMesh(core_axis_name="c", subcore_axis_name="s")


def _unpk(word_i32):
    lo = plsc.bitcast(lax.shift_left(word_i32, 16), jnp.float32)
    hi = plsc.bitcast(word_i32, jnp.float32)
    return lo, hi


@functools.partial(
    pl.kernel,
    out_type=[
        jax.ShapeDtypeStruct((B,), jnp.float32),
        jax.ShapeDtypeStruct((B,), jnp.float32),
        jax.ShapeDtypeStruct((B * NNEG,), jnp.float32),
        jax.ShapeDtypeStruct((B,), jnp.float32),
        jax.ShapeDtypeStruct((B * NNEG,), jnp.float32),
    ],
    mesh=_sc_mesh,
    compiler_params=pltpu.CompilerParams(needs_layout_passes=False),
    scratch_types=[
        pltpu.VMEM((BPW,), jnp.int32),
        pltpu.VMEM((BPW,), jnp.int32),
        pltpu.VMEM((BPW * NNEG,), jnp.int32),
        pltpu.VMEM((C, DW), jnp.int32),
        pltpu.VMEM((C, DW), jnp.int32),
        pltpu.VMEM((C, DW), jnp.int32),
        pltpu.VMEM((C, DW), jnp.int32),
        pltpu.VMEM((CN, DW), jnp.int32),
        pltpu.VMEM((CN, DW), jnp.int32),
        pltpu.VMEM((BPW,), jnp.float32),
        pltpu.VMEM((BPW,), jnp.float32),
        pltpu.VMEM((BPW * NNEG,), jnp.float32),
        pltpu.VMEM((BPW,), jnp.float32),
        pltpu.VMEM((BPW * NNEG,), jnp.float32),
        pltpu.SemaphoreType.DMA,
        pltpu.SemaphoreType.DMA,
        pltpu.SemaphoreType.DMA,
    ],
)
def _sc_volumes(pos_u_h, pos_w_h, negf_h, zzw_h, zzc_h, vw_h, vc_h,
                tv_h, pv_h, nv_h, pi_h, ni_h,
                idx_u, idx_w, idx_n, rows_u0, rows_u1,
                rows_w0, rows_w1, rows_n0, rows_n1,
                o_tv, o_pv, o_nv, o_pi, o_ni, sem0, sem1, semv):
    wid = lax.axis_index("c") * NS + lax.axis_index("s")
    base = wid * BPW

    pltpu.sync_copy(pos_u_h.at[pl.ds(base, BPW)], idx_u)
    pltpu.sync_copy(pos_w_h.at[pl.ds(base, BPW)], idx_w)
    pltpu.sync_copy(negf_h.at[pl.ds(base * NNEG, BPW * NNEG)], idx_n)

    for g4 in range(BPW // 128):
        pltpu.async_copy(vw_h.at[idx_u.at[pl.ds(g4 * 128, 128)]],
                         o_tv.at[pl.ds(g4 * 128, 128)], semv)
        pltpu.async_copy(vc_h.at[idx_w.at[pl.ds(g4 * 128, 128)]],
                         o_pv.at[pl.ds(g4 * 128, 128)], semv)

    def vol_issue(gi, _):
        voff = pl.multiple_of(gi * 128, 8)
        pltpu.async_copy(vc_h.at[idx_n.at[pl.ds(voff, 128)]],
                         o_nv.at[pl.ds(voff, 128)], semv)
        return 0

    lax.fori_loop(0, BPW * NNEG // 128, vol_issue, 0)

    sems = (sem0, sem1)
    rows_u = (rows_u0, rows_u1)
    rows_w = (rows_w0, rows_w1)
    rows_n = (rows_n0, rows_n1)
    zero16 = jnp.zeros((16,), jnp.float32)
    lanes = lax.iota(jnp.int32, 16)

    def issue(ci, sl):
        sem = sems[sl]
        off = pl.multiple_of(ci * C, 8)
        noff = pl.multiple_of(ci * CN, 8)
        pltpu.async_copy(zzw_h.at[idx_u.at[pl.ds(off, C)]],
                         rows_u[sl], sem)
        pltpu.async_copy(zzc_h.at[idx_w.at[pl.ds(off, C)]],
                         rows_w[sl], sem)
        pltpu.async_copy(zzc_h.at[idx_n.at[pl.ds(noff, 128)]],
                         rows_n[sl].at[pl.ds(0, 128)], sem)
        pltpu.async_copy(zzc_h.at[idx_n.at[pl.ds(noff + 128, 128)]],
                         rows_n[sl].at[pl.ds(128, 128)], sem)
        pltpu.async_copy(zzc_h.at[idx_n.at[pl.ds(noff + 256, CN - 256)]],
                         rows_n[sl].at[pl.ds(256, CN - 256)], sem)

    def drain(sl):
        sem = sems[sl]
        pltpu.make_async_copy(zzw_h.at[pl.ds(0, C)], rows_u[sl], sem).wait()
        pltpu.make_async_copy(zzc_h.at[pl.ds(0, C)], rows_w[sl], sem).wait()
        pltpu.make_async_copy(zzc_h.at[pl.ds(0, CN)], rows_n[sl], sem).wait()

    def compute(ci, sl):
        ru = rows_u[sl]
        rw = rows_w[sl]
        rn = rows_n[sl]

        def pos_d(d, pi_):
            dv = jnp.bitwise_and(d + lanes, DH - 1)
            uzw = plsc.load_gather(ru, [lanes, dv])
            uZw = plsc.load_gather(ru, [lanes, dv + DH])
            wzw = plsc.load_gather(rw, [lanes, dv])
            wZw = plsc.load_gather(rw, [lanes, dv + DH])
            uzl, uzh = _unpk(uzw)
            uZl, uZh = _unpk(uZw)
            wzl, wzh = _unpk(wzw)
            wZl, wZh = _unpk(wZw)
            pi_ = (pi_
                   + _logsp(jnp.minimum(wZl, uZl) - jnp.maximum(wzl, uzl))
                   + _logsp(jnp.minimum(wZh, uZh) - jnp.maximum(wzh, uzh)))
            return pi_

        pi_ = lax.fori_loop(0, DH, pos_d, zero16, unroll=4)
        eoff = pl.multiple_of(ci * C, 8)
        o_pi[pl.ds(eoff, 16)] = pi_

        def grp(g, _):
            rvec = lanes + g * 16
            evec = lax.div(rvec, jnp.int32(NNEG))

            def neg_d(d, ni):
                dv = jnp.bitwise_and(d + lanes, DH - 1)
                nzw = plsc.load_gather(rn, [rvec, dv])
                nZw = plsc.load_gather(rn, [rvec, dv + DH])
                uzw = plsc.load_gather(ru, [evec, dv])
                uZw = plsc.load_gather(ru, [evec, dv + DH])
                nzl, nzh = _unpk(nzw)
                nZl, nZh = _unpk(nZw)
                uzl, uzh = _unpk(uzw)
                uZl, uZh = _unpk(uZw)
                ni = (ni
                      + _logsp(jnp.minimum(nZl, uZl) - jnp.maximum(nzl, uzl))
                      + _logsp(jnp.minimum(nZh, uZh) - jnp.maximum(nzh, uzh)))
                return ni

            ni = lax.fori_loop(0, DH, neg_d, zero16, unroll=4)
            foff = pl.multiple_of(ci * CN + g * 16, 8)
            o_ni[pl.ds(foff, 16)] = ni
            return 0

        lax.fori_loop(0, NG, grp, 0)

    issue(0, 0)

    def outer(p, _):
        for b in range(2):
            ci = p * 2 + b

            @pl.when(ci + 1 < NCHUNK)
            def _():
                issue(ci + 1, 1 - b)

            drain(b)
            compute(ci, b)
        return 0

    lax.fori_loop(0, NCHUNK // 2, outer, 0)

    pltpu.make_async_copy(vw_h.at[pl.ds(0, BPW)], o_tv, semv).wait()
    pltpu.make_async_copy(vw_h.at[pl.ds(0, BPW)], o_pv, semv).wait()
    pltpu.make_async_copy(vc_h.at[pl.ds(0, BPW * NNEG)], o_nv, semv).wait()

    pltpu.sync_copy(o_tv, tv_h.at[pl.ds(base, BPW)])
    pltpu.sync_copy(o_pv, pv_h.at[pl.ds(base, BPW)])
    pltpu.sync_copy(o_nv, nv_h.at[pl.ds(base * NNEG, BPW * NNEG)])
    pltpu.sync_copy(o_pi, pi_h.at[pl.ds(base, BPW)])
    pltpu.sync_copy(o_ni, ni_h.at[pl.ds(base * NNEG, BPW * NNEG)])


def kernel(pos_u, pos_w, neg_w, W_word, W_ctx):
    zzw, zzc, vw, vc = _transform_tables(W_word, W_ctx)
    neg_flat = neg_w.reshape(-1)
    tv, pv, nvf, pi, nif = _sc_volumes(
        pos_u, pos_w, neg_flat, zzw, zzc, vw.reshape(V), vc.reshape(V))
    return (tv, pv, nvf.reshape(B, NNEG), pi, nif.reshape(B, NNEG))


# speedup vs baseline: 1.3435x; 1.0512x over previous
import functools

import jax
import jax.numpy as jnp
from jax import lax
from jax.experimental import pallas as pl
from jax.experimental.pallas import tpu as pltpu
from jax.experimental.pallas import tpu_sc as plsc

V = 100000
D = 128
D2 = 2 * D
DW = D
DH = D // 2
B = 16384
NNEG = 20
NC, NS = 2, 16
NW = NC * NS
BPW = B // NW
C = 16
CN = C * NNEG
NG = CN // 16
NCHUNK = BPW // C

_PC = (
    0.00022841986642679486,
    0.002200562996050353,
    -0.0049591490971503671,
    -0.079778088603572717,
    0.72134636444934774,
    -0.36651556254295736,
)


def _logsp(t):
    r = t * _PC[0] + _PC[1]
    for c in _PC[2:]:
        r = r * t + c
    return r


_TX_ROWS = 2000


def _bf16_code(x):
    code = lax.bitcast_convert_type(x.astype(jnp.bfloat16), jnp.uint16)
    return code.astype(jnp.uint32)


def _pack_half(h):
    lo = _bf16_code(h[:, :DH])
    hi = _bf16_code(h[:, DH:])
    return lax.bitcast_convert_type(lo | (hi << 16), jnp.int32)


def _tx_one(vec):
    w = vec[:, :D]
    Wc = vec[:, D:]
    z = jax.nn.sigmoid(w)
    Z = z + jax.nn.sigmoid(Wc) * (1.0 - z)
    packed = jnp.concatenate([_pack_half(z), _pack_half(Z)], axis=1)
    vol = jnp.sum(_logsp(Z - z), axis=-1, keepdims=True)
    return packed, vol


def _tx_kernel(word_ref, ctx_ref, ow_ref, oc_ref, vw_ref, vc_ref):
    ow_ref[...], vw_ref[...] = _tx_one(word_ref[...])
    oc_ref[...], vc_ref[...] = _tx_one(ctx_ref[...])


def _transform_tables(W_word, W_ctx):
    in_spec = pl.BlockSpec((_TX_ROWS, D2), lambda i: (i, 0))
    out_spec = pl.BlockSpec((_TX_ROWS, DW), lambda i: (i, 0))
    vol_spec = pl.BlockSpec((_TX_ROWS, 1), lambda i: (i, 0))
    return pl.pallas_call(
        _tx_kernel,
        grid=(V // _TX_ROWS,),
        in_specs=[in_spec, in_spec],
        out_specs=[out_spec, out_spec, vol_spec, vol_spec],
        out_shape=[jax.ShapeDtypeStruct((V, DW), jnp.int32)] * 2
        + [jax.ShapeDtypeStruct((V, 1), jnp.float32)] * 2,
    )(W_word, W_ctx)


_sc_mesh = plsc.VectorSubcoreMesh(core_axis_name="c", subcore_axis_name="s")


def _unpk(word_i32):
    lo = plsc.bitcast(lax.shift_left(word_i32, 16), jnp.float32)
    hi = plsc.bitcast(word_i32, jnp.float32)
    return lo, hi


@functools.partial(
    pl.kernel,
    out_type=[
        jax.ShapeDtypeStruct((B,), jnp.float32),
        jax.ShapeDtypeStruct((B,), jnp.float32),
        jax.ShapeDtypeStruct((B * NNEG,), jnp.float32),
        jax.ShapeDtypeStruct((B,), jnp.float32),
        jax.ShapeDtypeStruct((B * NNEG,), jnp.float32),
    ],
    mesh=_sc_mesh,
    compiler_params=pltpu.CompilerParams(needs_layout_passes=False),
    scratch_types=[
        pltpu.VMEM((BPW,), jnp.int32),
        pltpu.VMEM((BPW,), jnp.int32),
        pltpu.VMEM((BPW * NNEG,), jnp.int32),
        pltpu.VMEM((C, DW), jnp.int32),
        pltpu.VMEM((C, DW), jnp.int32),
        pltpu.VMEM((C, DW), jnp.int32),
        pltpu.VMEM((C, DW), jnp.int32),
        pltpu.VMEM((CN, DW), jnp.int32),
        pltpu.VMEM((CN, DW), jnp.int32),
        pltpu.VMEM((BPW,), jnp.float32),
        pltpu.VMEM((BPW,), jnp.float32),
        pltpu.VMEM((BPW * NNEG,), jnp.float32),
        pltpu.VMEM((BPW,), jnp.float32),
        pltpu.VMEM((BPW * NNEG,), jnp.float32),
        pltpu.SemaphoreType.DMA,
        pltpu.SemaphoreType.DMA,
        pltpu.SemaphoreType.DMA,
    ],
)
def _sc_volumes(pos_u_h, pos_w_h, negf_h, zzw_h, zzc_h, vw_h, vc_h,
                tv_h, pv_h, nv_h, pi_h, ni_h,
                idx_u, idx_w, idx_n, rows_u0, rows_u1,
                rows_w0, rows_w1, rows_n0, rows_n1,
                o_tv, o_pv, o_nv, o_pi, o_ni, sem0, sem1, semv):
    wid = lax.axis_index("c") * NS + lax.axis_index("s")
    base = wid * BPW

    pltpu.sync_copy(pos_u_h.at[pl.ds(base, BPW)], idx_u)
    pltpu.sync_copy(pos_w_h.at[pl.ds(base, BPW)], idx_w)
    pltpu.sync_copy(negf_h.at[pl.ds(base * NNEG, BPW * NNEG)], idx_n)

    for g4 in range(BPW // 128):
        pltpu.async_copy(vw_h.at[idx_u.at[pl.ds(g4 * 128, 128)]],
                         o_tv.at[pl.ds(g4 * 128, 128)], semv)
        pltpu.async_copy(vc_h.at[idx_w.at[pl.ds(g4 * 128, 128)]],
                         o_pv.at[pl.ds(g4 * 128, 128)], semv)

    def vol_issue(gi, _):
        voff = pl.multiple_of(gi * 128, 8)
        pltpu.async_copy(vc_h.at[idx_n.at[pl.ds(voff, 128)]],
                         o_nv.at[pl.ds(voff, 128)], semv)
        return 0

    lax.fori_loop(0, BPW * NNEG // 128, vol_issue, 0)

    sems = (sem0, sem1)
    rows_u = (rows_u0, rows_u1)
    rows_w = (rows_w0, rows_w1)
    rows_n = (rows_n0, rows_n1)
    zero16 = jnp.zeros((16,), jnp.float32)
    lanes = lax.iota(jnp.int32, 16)

    def issue(ci, sl):
        sem = sems[sl]
        off = pl.multiple_of(ci * C, 8)
        noff = pl.multiple_of(ci * CN, 8)
        pltpu.async_copy(zzw_h.at[idx_u.at[pl.ds(off, C)]],
                         rows_u[sl], sem)
        pltpu.async_copy(zzc_h.at[idx_w.at[pl.ds(off, C)]],
                         rows_w[sl], sem)
        pltpu.async_copy(zzc_h.at[idx_n.at[pl.ds(noff, 128)]],
                         rows_n[sl].at[pl.ds(0, 128)], sem)
        pltpu.async_copy(zzc_h.at[idx_n.at[pl.ds(noff + 128, 128)]],
                         rows_n[sl].at[pl.ds(128, 128)], sem)
        pltpu.async_copy(zzc_h.at[idx_n.at[pl.ds(noff + 256, CN - 256)]],
                         rows_n[sl].at[pl.ds(256, CN - 256)], sem)

    def drain(sl):
        sem = sems[sl]
        pltpu.make_async_copy(zzw_h.at[pl.ds(0, C)], rows_u[sl], sem).wait()
        pltpu.make_async_copy(zzc_h.at[pl.ds(0, C)], rows_w[sl], sem).wait()
        pltpu.make_async_copy(zzc_h.at[pl.ds(0, CN)], rows_n[sl], sem).wait()

    def compute(ci, sl):
        ru = rows_u[sl]
        rw = rows_w[sl]
        rn = rows_n[sl]

        def pos_d(d, pi_):
            dv = jnp.bitwise_and(d + lanes, DH - 1)
            uzw = plsc.load_gather(ru, [lanes, dv])
            uZw = plsc.load_gather(ru, [lanes, dv + DH])
            wzw = plsc.load_gather(rw, [lanes, dv])
            wZw = plsc.load_gather(rw, [lanes, dv + DH])
            uzl, uzh = _unpk(uzw)
            uZl, uZh = _unpk(uZw)
            wzl, wzh = _unpk(wzw)
            wZl, wZh = _unpk(wZw)
            pi_ = (pi_
                   + _logsp(jnp.minimum(wZl, uZl) - jnp.maximum(wzl, uzl))
                   + _logsp(jnp.minimum(wZh, uZh) - jnp.maximum(wzh, uzh)))
            return pi_

        pi_ = lax.fori_loop(0, DH, pos_d, zero16, unroll=4)
        eoff = pl.multiple_of(ci * C, 8)
        o_pi[pl.ds(eoff, 16)] = pi_

        def grp(g, _):
            rvec = lanes + g * 16
            evec = lax.div(rvec, jnp.int32(NNEG))

            def neg_d(d, ni):
                dv = jnp.bitwise_and(d + lanes, DH - 1)
                nzw = plsc.load_gather(rn, [rvec, dv])
                nZw = plsc.load_gather(rn, [rvec, dv + DH])
                uzw = plsc.load_gather(ru, [evec, dv])
                uZw = plsc.load_gather(ru, [evec, dv + DH])
                nzl, nzh = _unpk(nzw)
                nZl, nZh = _unpk(nZw)
                uzl, uzh = _unpk(uzw)
                uZl, uZh = _unpk(uZw)
                ni = (ni
                      + _logsp(jnp.minimum(nZl, uZl) - jnp.maximum(nzl, uzl))
                      + _logsp(jnp.minimum(nZh, uZh) - jnp.maximum(nzh, uzh)))
                return ni

            ni = lax.fori_loop(0, DH, neg_d, zero16, unroll=4)
            foff = pl.multiple_of(ci * CN + g * 16, 8)
            o_ni[pl.ds(foff, 16)] = ni
            return 0

        lax.fori_loop(0, NG, grp, 0)

    issue(0, 0)

    def outer(p, _):
        for b in range(2):
            ci = p * 2 + b

            @pl.when(ci + 1 < NCHUNK)
            def _():
                issue(ci + 1, 1 - b)

            drain(b)
            compute(ci, b)
        return 0

    lax.fori_loop(0, NCHUNK // 2, outer, 0)

    pltpu.make_async_copy(vw_h.at[pl.ds(0, BPW)], o_tv, semv).wait()
    pltpu.make_async_copy(vw_h.at[pl.ds(0, BPW)], o_pv, semv).wait()
    pltpu.make_async_copy(vc_h.at[pl.ds(0, BPW * NNEG)], o_nv, semv).wait()

    pltpu.sync_copy(o_tv, tv_h.at[pl.ds(base, BPW)])
    pltpu.sync_copy(o_pv, pv_h.at[pl.ds(base, BPW)])
    pltpu.sync_copy(o_nv, nv_h.at[pl.ds(base * NNEG, BPW * NNEG)])
    pltpu.sync_copy(o_pi, pi_h.at[pl.ds(base, BPW)])
    pltpu.sync_copy(o_ni, ni_h.at[pl.ds(base * NNEG, BPW * NNEG)])


def kernel(pos_u, pos_w, neg_w, W_word, W_ctx):
    zzw, zzc, vw, vc = _transform_tables(W_word, W_ctx)
    neg_flat = neg_w.reshape(-1)
    tv, pv, nvf, pi, nif = _sc_volumes(
        pos_u, pos_w, neg_flat, zzw, zzc, vw.reshape(V), vc.reshape(V))
    return (tv, pv, nvf.reshape(B, NNEG), pi, nif.reshape(B, NNEG))


# speedup vs baseline: 1.6941x; 1.2609x over previous
import functools

import jax
import jax.numpy as jnp
from jax import lax
from jax.experimental import pallas as pl
from jax.experimental.pallas import tpu as pltpu
from jax.experimental.pallas import tpu_sc as plsc

V = 100000
D = 128
D2 = 2 * D
DW = D
DH = D // 2
B = 16384
NNEG = 20
NC, NS = 2, 16
NW = NC * NS
BPW = B // NW
C = 16
CN = C * NNEG
NG = CN // 16
NCHUNK = BPW // C

_PC = (
    0.00022841986642679486,
    0.002200562996050353,
    -0.0049591490971503671,
    -0.079778088603572717,
    0.72134636444934774,
    -0.36651556254295736,
)


def _logsp(t):
    r = t * _PC[0] + _PC[1]
    for c in _PC[2:]:
        r = r * t + c
    return r


_TX_ROWS = 2000


def _bf16_code(x):
    code = lax.bitcast_convert_type(x.astype(jnp.bfloat16), jnp.uint16)
    return code.astype(jnp.uint32)


def _pack_half(h):
    lo = _bf16_code(h[:, :DH])
    hi = _bf16_code(h[:, DH:])
    return lax.bitcast_convert_type(lo | (hi << 16), jnp.int32)


def _tx_one(vec):
    w = vec[:, :D]
    Wc = vec[:, D:]
    z = jax.nn.sigmoid(w)
    Z = z + jax.nn.sigmoid(Wc) * (1.0 - z)
    packed = jnp.concatenate([_pack_half(z), _pack_half(Z)], axis=1)
    vol = jnp.sum(_logsp(Z - z), axis=-1, keepdims=True)
    return packed, vol


def _tx_kernel(word_ref, ctx_ref, ow_ref, oc_ref, vw_ref, vc_ref):
    ow_ref[...], vw_ref[...] = _tx_one(word_ref[...])
    oc_ref[...], vc_ref[...] = _tx_one(ctx_ref[...])


def _transform_tables(W_word, W_ctx):
    in_spec = pl.BlockSpec((_TX_ROWS, D2), lambda i: (i, 0))
    out_spec = pl.BlockSpec((_TX_ROWS, DW), lambda i: (i, 0))
    vol_spec = pl.BlockSpec((_TX_ROWS, 1), lambda i: (i, 0))
    return pl.pallas_call(
        _tx_kernel,
        grid=(V // _TX_ROWS,),
        in_specs=[in_spec, in_spec],
        out_specs=[out_spec, out_spec, vol_spec, vol_spec],
        out_shape=[jax.ShapeDtypeStruct((V, DW), jnp.int32)] * 2
        + [jax.ShapeDtypeStruct((V, 1), jnp.float32)] * 2,
    )(W_word, W_ctx)


_sc_mesh = plsc.VectorSubcoreMesh(core_axis_name="c", subcore_axis_name="s")


def _bf(word_i32):
    return plsc.bitcast(word_i32, jnp.bfloat16)


def _logsp_bf(t):
    r = t * jnp.bfloat16(_PC[0]) + jnp.bfloat16(_PC[1])
    for c in _PC[2:]:
        r = r * t + jnp.bfloat16(c)
    return r


def _int_term(az_w, aZ_w, bz_w, bZ_w):
    t = jnp.minimum(_bf(aZ_w), _bf(bZ_w)) - jnp.maximum(_bf(az_w), _bf(bz_w))
    lo, hi = plsc.unpack(_logsp_bf(t), format=plsc.PackFormat.INTERLEAVED)
    return lo, hi


@functools.partial(
    pl.kernel,
    out_type=[
        jax.ShapeDtypeStruct((B,), jnp.float32),
        jax.ShapeDtypeStruct((B,), jnp.float32),
        jax.ShapeDtypeStruct((B * NNEG,), jnp.float32),
        jax.ShapeDtypeStruct((B,), jnp.float32),
        jax.ShapeDtypeStruct((B * NNEG,), jnp.float32),
    ],
    mesh=_sc_mesh,
    compiler_params=pltpu.CompilerParams(needs_layout_passes=False),
    scratch_types=[
        pltpu.VMEM((BPW,), jnp.int32),
        pltpu.VMEM((BPW,), jnp.int32),
        pltpu.VMEM((BPW * NNEG,), jnp.int32),
        pltpu.VMEM((C, DW), jnp.int32),
        pltpu.VMEM((C, DW), jnp.int32),
        pltpu.VMEM((C, DW), jnp.int32),
        pltpu.VMEM((C, DW), jnp.int32),
        pltpu.VMEM((CN, DW), jnp.int32),
        pltpu.VMEM((CN, DW), jnp.int32),
        pltpu.VMEM((BPW,), jnp.float32),
        pltpu.VMEM((BPW,), jnp.float32),
        pltpu.VMEM((BPW * NNEG,), jnp.float32),
        pltpu.VMEM((BPW,), jnp.float32),
        pltpu.VMEM((BPW * NNEG,), jnp.float32),
        pltpu.SemaphoreType.DMA,
        pltpu.SemaphoreType.DMA,
        pltpu.SemaphoreType.DMA,
    ],
)
def _sc_volumes(pos_u_h, pos_w_h, negf_h, zzw_h, zzc_h, vw_h, vc_h,
                tv_h, pv_h, nv_h, pi_h, ni_h,
                idx_u, idx_w, idx_n, rows_u0, rows_u1,
                rows_w0, rows_w1, rows_n0, rows_n1,
                o_tv, o_pv, o_nv, o_pi, o_ni, sem0, sem1, semv):
    wid = lax.axis_index("c") * NS + lax.axis_index("s")
    base = wid * BPW

    pltpu.sync_copy(pos_u_h.at[pl.ds(base, BPW)], idx_u)
    pltpu.sync_copy(pos_w_h.at[pl.ds(base, BPW)], idx_w)
    pltpu.sync_copy(negf_h.at[pl.ds(base * NNEG, BPW * NNEG)], idx_n)

    for g4 in range(BPW // 128):
        pltpu.async_copy(vw_h.at[idx_u.at[pl.ds(g4 * 128, 128)]],
                         o_tv.at[pl.ds(g4 * 128, 128)], semv)
        pltpu.async_copy(vc_h.at[idx_w.at[pl.ds(g4 * 128, 128)]],
                         o_pv.at[pl.ds(g4 * 128, 128)], semv)

    def vol_issue(gi, _):
        voff = pl.multiple_of(gi * 128, 8)
        pltpu.async_copy(vc_h.at[idx_n.at[pl.ds(voff, 128)]],
                         o_nv.at[pl.ds(voff, 128)], semv)
        return 0

    lax.fori_loop(0, BPW * NNEG // 128, vol_issue, 0)

    sems = (sem0, sem1)
    rows_u = (rows_u0, rows_u1)
    rows_w = (rows_w0, rows_w1)
    rows_n = (rows_n0, rows_n1)
    zero16 = jnp.zeros((16,), jnp.float32)
    lanes = lax.iota(jnp.int32, 16)

    def issue(ci, sl):
        sem = sems[sl]
        off = pl.multiple_of(ci * C, 8)
        noff = pl.multiple_of(ci * CN, 8)
        pltpu.async_copy(zzw_h.at[idx_u.at[pl.ds(off, C)]],
                         rows_u[sl], sem)
        pltpu.async_copy(zzc_h.at[idx_w.at[pl.ds(off, C)]],
                         rows_w[sl], sem)
        pltpu.async_copy(zzc_h.at[idx_n.at[pl.ds(noff, 128)]],
                         rows_n[sl].at[pl.ds(0, 128)], sem)
        pltpu.async_copy(zzc_h.at[idx_n.at[pl.ds(noff + 128, 128)]],
                         rows_n[sl].at[pl.ds(128, 128)], sem)
        pltpu.async_copy(zzc_h.at[idx_n.at[pl.ds(noff + 256, CN - 256)]],
                         rows_n[sl].at[pl.ds(256, CN - 256)], sem)

    def drain(sl):
        sem = sems[sl]
        pltpu.make_async_copy(zzw_h.at[pl.ds(0, C)], rows_u[sl], sem).wait()
        pltpu.make_async_copy(zzc_h.at[pl.ds(0, C)], rows_w[sl], sem).wait()
        pltpu.make_async_copy(zzc_h.at[pl.ds(0, CN)], rows_n[sl], sem).wait()

    def compute(ci, sl):
        ru = rows_u[sl]
        rw = rows_w[sl]
        rn = rows_n[sl]

        def pos_d(d, pi_):
            dv = jnp.bitwise_and(d + lanes, DH - 1)
            uzw = plsc.load_gather(ru, [lanes, dv])
            uZw = plsc.load_gather(ru, [lanes, dv + DH])
            wzw = plsc.load_gather(rw, [lanes, dv])
            wZw = plsc.load_gather(rw, [lanes, dv + DH])
            lo, hi = _int_term(uzw, uZw, wzw, wZw)
            return pi_ + lo + hi

        pi_ = lax.fori_loop(0, DH, pos_d, zero16, unroll=4)
        eoff = pl.multiple_of(ci * C, 8)
        o_pi[pl.ds(eoff, 16)] = pi_

        def grp(g, _):
            rvec = lanes + g * 16
            evec = lax.div(rvec, jnp.int32(NNEG))

            def neg_d(d, ni):
                dv = jnp.bitwise_and(d + lanes, DH - 1)
                nzw = plsc.load_gather(rn, [rvec, dv])
                nZw = plsc.load_gather(rn, [rvec, dv + DH])
                uzw = plsc.load_gather(ru, [evec, dv])
                uZw = plsc.load_gather(ru, [evec, dv + DH])
                lo, hi = _int_term(nzw, nZw, uzw, uZw)
                return ni + lo + hi

            ni = lax.fori_loop(0, DH, neg_d, zero16, unroll=4)
            foff = pl.multiple_of(ci * CN + g * 16, 8)
            o_ni[pl.ds(foff, 16)] = ni
            return 0

        lax.fori_loop(0, NG, grp, 0)

    issue(0, 0)

    def outer(p, _):
        for b in range(2):
            ci = p * 2 + b

            @pl.when(ci + 1 < NCHUNK)
            def _():
                issue(ci + 1, 1 - b)

            drain(b)
            compute(ci, b)
        return 0

    lax.fori_loop(0, NCHUNK // 2, outer, 0)

    pltpu.make_async_copy(vw_h.at[pl.ds(0, BPW)], o_tv, semv).wait()
    pltpu.make_async_copy(vw_h.at[pl.ds(0, BPW)], o_pv, semv).wait()
    pltpu.make_async_copy(vc_h.at[pl.ds(0, BPW * NNEG)], o_nv, semv).wait()

    pltpu.sync_copy(o_tv, tv_h.at[pl.ds(base, BPW)])
    pltpu.sync_copy(o_pv, pv_h.at[pl.ds(base, BPW)])
    pltpu.sync_copy(o_nv, nv_h.at[pl.ds(base * NNEG, BPW * NNEG)])
    pltpu.sync_copy(o_pi, pi_h.at[pl.ds(base, BPW)])
    pltpu.sync_copy(o_ni, ni_h.at[pl.ds(base * NNEG, BPW * NNEG)])


def kernel(pos_u, pos_w, neg_w, W_word, W_ctx):
    zzw, zzc, vw, vc = _transform_tables(W_word, W_ctx)
    neg_flat = neg_w.reshape(-1)
    tv, pv, nvf, pi, nif = _sc_volumes(
        pos_u, pos_w, neg_flat, zzw, zzc, vw.reshape(V), vc.reshape(V))
    return (tv, pv, nvf.reshape(B, NNEG), pi, nif.reshape(B, NNEG))


# speedup vs baseline: 1.7495x; 1.0327x over previous
import functools

import jax
import jax.numpy as jnp
from jax import lax
from jax.experimental import pallas as pl
from jax.experimental.pallas import tpu as pltpu
from jax.experimental.pallas import tpu_sc as plsc

V = 100000
D = 128
D2 = 2 * D
DW = D
DH = D // 2
B = 16384
NNEG = 20
NC, NS = 2, 16
NW = NC * NS
BPW = B // NW
C = 16
CN = C * NNEG
NG = CN // 16
NCHUNK = BPW // C

_PC = (
    0.00022841986642679486,
    0.002200562996050353,
    -0.0049591490971503671,
    -0.079778088603572717,
    0.72134636444934774,
    -0.36651556254295736,
)


def _logsp(t):
    r = t * _PC[0] + _PC[1]
    for c in _PC[2:]:
        r = r * t + c
    return r


_TX_ROWS = 2000


def _bf16_code(x):
    code = lax.bitcast_convert_type(x.astype(jnp.bfloat16), jnp.uint16)
    return code.astype(jnp.uint32)


def _pack_half(h):
    lo = _bf16_code(h[:, :DH])
    hi = _bf16_code(h[:, DH:])
    return lax.bitcast_convert_type(lo | (hi << 16), jnp.int32)


def _tx_one(vec):
    w = vec[:, :D]
    Wc = vec[:, D:]
    z = jax.nn.sigmoid(w)
    Z = z + jax.nn.sigmoid(Wc) * (1.0 - z)
    packed = jnp.concatenate([_pack_half(z), _pack_half(Z)], axis=1)
    vol = jnp.sum(jnp.log(jnp.log1p(jnp.exp(Z - z))), axis=-1, keepdims=True)
    return packed, vol


def _tx_kernel(word_ref, ctx_ref, ow_ref, oc_ref, vw_ref, vc_ref):
    ow_ref[...], vw_ref[...] = _tx_one(word_ref[...])
    oc_ref[...], vc_ref[...] = _tx_one(ctx_ref[...])


def _transform_tables(W_word, W_ctx):
    in_spec = pl.BlockSpec((_TX_ROWS, D2), lambda i: (i, 0))
    out_spec = pl.BlockSpec((_TX_ROWS, DW), lambda i: (i, 0))
    vol_spec = pl.BlockSpec((_TX_ROWS, 1), lambda i: (i, 0))
    return pl.pallas_call(
        _tx_kernel,
        grid=(V // _TX_ROWS,),
        in_specs=[in_spec, in_spec],
        out_specs=[out_spec, out_spec, vol_spec, vol_spec],
        out_shape=[jax.ShapeDtypeStruct((V, DW), jnp.int32)] * 2
        + [jax.ShapeDtypeStruct((V, 1), jnp.float32)] * 2,
    )(W_word, W_ctx)


_sc_mesh = plsc.VectorSubcoreMesh(core_axis_name="c", subcore_axis_name="s")


def _bf(word_i32):
    return plsc.bitcast(word_i32, jnp.bfloat16)


_PC4 = (
    0.0022005629960468935,
    -0.0047053479765727345,
    -0.079778088603569511,
    0.72129197822300972,
    -0.36651556254295792,
)


def _logsp_bf(t):
    r = t * jnp.bfloat16(_PC4[0]) + jnp.bfloat16(_PC4[1])
    for c in _PC4[2:]:
        r = r * t + jnp.bfloat16(c)
    return r


def _int_term(az_w, aZ_w, bz_w, bZ_w):
    t = jnp.minimum(_bf(aZ_w), _bf(bZ_w)) - jnp.maximum(_bf(az_w), _bf(bz_w))
    lo, hi = plsc.unpack(_logsp_bf(t), format=plsc.PackFormat.INTERLEAVED)
    return lo, hi


@functools.partial(
    pl.kernel,
    out_type=[
        jax.ShapeDtypeStruct((B,), jnp.float32),
        jax.ShapeDtypeStruct((B,), jnp.float32),
        jax.ShapeDtypeStruct((B * NNEG,), jnp.float32),
        jax.ShapeDtypeStruct((B,), jnp.float32),
        jax.ShapeDtypeStruct((B * NNEG,), jnp.float32),
    ],
    mesh=_sc_mesh,
    compiler_params=pltpu.CompilerParams(needs_layout_passes=False),
    scratch_types=[
        pltpu.VMEM((BPW,), jnp.int32),
        pltpu.VMEM((BPW,), jnp.int32),
        pltpu.VMEM((BPW * NNEG,), jnp.int32),
        pltpu.VMEM((C, DW), jnp.int32),
        pltpu.VMEM((C, DW), jnp.int32),
        pltpu.VMEM((C, DW), jnp.int32),
        pltpu.VMEM((C, DW), jnp.int32),
        pltpu.VMEM((CN, DW), jnp.int32),
        pltpu.VMEM((CN, DW), jnp.int32),
        pltpu.VMEM((BPW,), jnp.float32),
        pltpu.VMEM((BPW,), jnp.float32),
        pltpu.VMEM((BPW * NNEG,), jnp.float32),
        pltpu.VMEM((BPW,), jnp.float32),
        pltpu.VMEM((BPW * NNEG,), jnp.float32),
        pltpu.SemaphoreType.DMA,
        pltpu.SemaphoreType.DMA,
        pltpu.SemaphoreType.DMA,
    ],
)
def _sc_volumes(pos_u_h, pos_w_h, negf_h, zzw_h, zzc_h, vw_h, vc_h,
                tv_h, pv_h, nv_h, pi_h, ni_h,
                idx_u, idx_w, idx_n, rows_u0, rows_u1,
                rows_w0, rows_w1, rows_n0, rows_n1,
                o_tv, o_pv, o_nv, o_pi, o_ni, sem0, sem1, semv):
    wid = lax.axis_index("c") * NS + lax.axis_index("s")
    base = wid * BPW

    pltpu.sync_copy(pos_u_h.at[pl.ds(base, BPW)], idx_u)
    pltpu.sync_copy(pos_w_h.at[pl.ds(base, BPW)], idx_w)
    pltpu.sync_copy(negf_h.at[pl.ds(base * NNEG, BPW * NNEG)], idx_n)

    for g4 in range(BPW // 128):
        pltpu.async_copy(vw_h.at[idx_u.at[pl.ds(g4 * 128, 128)]],
                         o_tv.at[pl.ds(g4 * 128, 128)], semv)
        pltpu.async_copy(vc_h.at[idx_w.at[pl.ds(g4 * 128, 128)]],
                         o_pv.at[pl.ds(g4 * 128, 128)], semv)

    def vol_issue(gi, _):
        voff = pl.multiple_of(gi * 128, 8)
        pltpu.async_copy(vc_h.at[idx_n.at[pl.ds(voff, 128)]],
                         o_nv.at[pl.ds(voff, 128)], semv)
        return 0

    lax.fori_loop(0, BPW * NNEG // 128, vol_issue, 0)

    sems = (sem0, sem1)
    rows_u = (rows_u0, rows_u1)
    rows_w = (rows_w0, rows_w1)
    rows_n = (rows_n0, rows_n1)
    zero16 = jnp.zeros((16,), jnp.float32)
    lanes = lax.iota(jnp.int32, 16)

    def issue(ci, sl):
        sem = sems[sl]
        off = pl.multiple_of(ci * C, 8)
        noff = pl.multiple_of(ci * CN, 8)
        pltpu.async_copy(zzw_h.at[idx_u.at[pl.ds(off, C)]],
                         rows_u[sl], sem)
        pltpu.async_copy(zzc_h.at[idx_w.at[pl.ds(off, C)]],
                         rows_w[sl], sem)
        pltpu.async_copy(zzc_h.at[idx_n.at[pl.ds(noff, 128)]],
                         rows_n[sl].at[pl.ds(0, 128)], sem)
        pltpu.async_copy(zzc_h.at[idx_n.at[pl.ds(noff + 128, 128)]],
                         rows_n[sl].at[pl.ds(128, 128)], sem)
        pltpu.async_copy(zzc_h.at[idx_n.at[pl.ds(noff + 256, CN - 256)]],
                         rows_n[sl].at[pl.ds(256, CN - 256)], sem)

    def drain(sl):
        sem = sems[sl]
        pltpu.make_async_copy(zzw_h.at[pl.ds(0, C)], rows_u[sl], sem).wait()
        pltpu.make_async_copy(zzc_h.at[pl.ds(0, C)], rows_w[sl], sem).wait()
        pltpu.make_async_copy(zzc_h.at[pl.ds(0, CN)], rows_n[sl], sem).wait()

    def compute(ci, sl):
        ru = rows_u[sl]
        rw = rows_w[sl]
        rn = rows_n[sl]

        def pos_d(d, pi_):
            dv = jnp.bitwise_and(d + lanes, DH - 1)
            uzw = plsc.load_gather(ru, [lanes, dv])
            uZw = plsc.load_gather(ru, [lanes, dv + DH])
            wzw = plsc.load_gather(rw, [lanes, dv])
            wZw = plsc.load_gather(rw, [lanes, dv + DH])
            lo, hi = _int_term(uzw, uZw, wzw, wZw)
            return pi_ + lo + hi

        pi_ = lax.fori_loop(0, DH, pos_d, zero16, unroll=4)
        eoff = pl.multiple_of(ci * C, 8)
        o_pi[pl.ds(eoff, 16)] = pi_

        def grp(g, _):
            rvec = lanes + g * 16
            evec = lax.div(rvec, jnp.int32(NNEG))

            def neg_d(d, ni):
                dv = jnp.bitwise_and(d + lanes, DH - 1)
                nzw = plsc.load_gather(rn, [rvec, dv])
                nZw = plsc.load_gather(rn, [rvec, dv + DH])
                uzw = plsc.load_gather(ru, [evec, dv])
                uZw = plsc.load_gather(ru, [evec, dv + DH])
                lo, hi = _int_term(nzw, nZw, uzw, uZw)
                return ni + lo + hi

            ni = lax.fori_loop(0, DH, neg_d, zero16, unroll=4)
            foff = pl.multiple_of(ci * CN + g * 16, 8)
            o_ni[pl.ds(foff, 16)] = ni
            return 0

        lax.fori_loop(0, NG, grp, 0)

    issue(0, 0)

    def outer(p, _):
        for b in range(2):
            ci = p * 2 + b

            @pl.when(ci + 1 < NCHUNK)
            def _():
                issue(ci + 1, 1 - b)

            drain(b)
            compute(ci, b)
        return 0

    lax.fori_loop(0, NCHUNK // 2, outer, 0)

    pltpu.make_async_copy(vw_h.at[pl.ds(0, BPW)], o_tv, semv).wait()
    pltpu.make_async_copy(vw_h.at[pl.ds(0, BPW)], o_pv, semv).wait()
    pltpu.make_async_copy(vc_h.at[pl.ds(0, BPW * NNEG)], o_nv, semv).wait()

    pltpu.sync_copy(o_tv, tv_h.at[pl.ds(base, BPW)])
    pltpu.sync_copy(o_pv, pv_h.at[pl.ds(base, BPW)])
    pltpu.sync_copy(o_nv, nv_h.at[pl.ds(base * NNEG, BPW * NNEG)])
    pltpu.sync_copy(o_pi, pi_h.at[pl.ds(base, BPW)])
    pltpu.sync_copy(o_ni, ni_h.at[pl.ds(base * NNEG, BPW * NNEG)])


def kernel(pos_u, pos_w, neg_w, W_word, W_ctx):
    zzw, zzc, vw, vc = _transform_tables(W_word, W_ctx)
    neg_flat = neg_w.reshape(-1)
    tv, pv, nvf, pi, nif = _sc_volumes(
        pos_u, pos_w, neg_flat, zzw, zzc, vw.reshape(V), vc.reshape(V))
    return (tv, pv, nvf.reshape(B, NNEG), pi, nif.reshape(B, NNEG))


# speedup vs baseline: 1.7955x; 1.0263x over previous
import functools

import jax
import jax.numpy as jnp
from jax import lax
from jax.experimental import pallas as pl
from jax.experimental.pallas import tpu as pltpu
from jax.experimental.pallas import tpu_sc as plsc

V = 100000
D = 128
D2 = 2 * D
DW = D
DH = D // 2
B = 16384
NNEG = 20
NC, NS = 2, 16
NW = NC * NS
BPW = B // NW
C = 16
CN = C * NNEG
NG = CN // 16
NCHUNK = BPW // C

_PC = (
    0.00022841986642679486,
    0.002200562996050353,
    -0.0049591490971503671,
    -0.079778088603572717,
    0.72134636444934774,
    -0.36651556254295736,
)


def _logsp(t):
    r = t * _PC[0] + _PC[1]
    for c in _PC[2:]:
        r = r * t + c
    return r


_TX_ROWS = 4000


def _bf16_code(x):
    code = lax.bitcast_convert_type(x.astype(jnp.bfloat16), jnp.uint16)
    return code.astype(jnp.uint32)


def _pack_half(h):
    lo = _bf16_code(h[:, :DH])
    hi = _bf16_code(h[:, DH:])
    return lax.bitcast_convert_type(lo | (hi << 16), jnp.int32)


def _tx_one(vec):
    w = vec[:, :D]
    Wc = vec[:, D:]
    z = jax.nn.sigmoid(w)
    Z = z + jax.nn.sigmoid(Wc) * (1.0 - z)
    packed = jnp.concatenate([_pack_half(z), _pack_half(Z)], axis=1)
    vol = jnp.sum(jnp.log(jnp.log1p(jnp.exp(Z - z))), axis=-1, keepdims=True)
    return packed, vol


def _tx_kernel(word_ref, ctx_ref, ow_ref, oc_ref, vw_ref, vc_ref):
    ow_ref[...], vw_ref[...] = _tx_one(word_ref[...])
    oc_ref[...], vc_ref[...] = _tx_one(ctx_ref[...])


def _transform_tables(W_word, W_ctx):
    in_spec = pl.BlockSpec((_TX_ROWS, D2), lambda i: (i, 0))
    out_spec = pl.BlockSpec((_TX_ROWS, DW), lambda i: (i, 0))
    vol_spec = pl.BlockSpec((_TX_ROWS, 1), lambda i: (i, 0))
    return pl.pallas_call(
        _tx_kernel,
        grid=(V // _TX_ROWS,),
        in_specs=[in_spec, in_spec],
        out_specs=[out_spec, out_spec, vol_spec, vol_spec],
        out_shape=[jax.ShapeDtypeStruct((V, DW), jnp.int32)] * 2
        + [jax.ShapeDtypeStruct((V, 1), jnp.float32)] * 2,
    )(W_word, W_ctx)


_sc_mesh = plsc.VectorSubcoreMesh(core_axis_name="c", subcore_axis_name="s")


def _bf(word_i32):
    return plsc.bitcast(word_i32, jnp.bfloat16)


_PC4 = (
    0.0022005629960468935,
    -0.0047053479765727345,
    -0.079778088603569511,
    0.72129197822300972,
    -0.36651556254295792,
)


def _logsp_bf(t):
    r = t * jnp.bfloat16(_PC4[0]) + jnp.bfloat16(_PC4[1])
    for c in _PC4[2:]:
        r = r * t + jnp.bfloat16(c)
    return r


def _int_term(az_w, aZ_w, bz_w, bZ_w):
    t = jnp.minimum(_bf(aZ_w), _bf(bZ_w)) - jnp.maximum(_bf(az_w), _bf(bz_w))
    lo, hi = plsc.unpack(_logsp_bf(t), format=plsc.PackFormat.INTERLEAVED)
    return lo, hi


@functools.partial(
    pl.kernel,
    out_type=[
        jax.ShapeDtypeStruct((B,), jnp.float32),
        jax.ShapeDtypeStruct((B,), jnp.float32),
        jax.ShapeDtypeStruct((B * NNEG,), jnp.float32),
        jax.ShapeDtypeStruct((B,), jnp.float32),
        jax.ShapeDtypeStruct((B * NNEG,), jnp.float32),
    ],
    mesh=_sc_mesh,
    compiler_params=pltpu.CompilerParams(needs_layout_passes=False),
    scratch_types=[
        pltpu.VMEM((BPW,), jnp.int32),
        pltpu.VMEM((BPW,), jnp.int32),
        pltpu.VMEM((BPW * NNEG,), jnp.int32),
        pltpu.VMEM((C, DW), jnp.int32),
        pltpu.VMEM((C, DW), jnp.int32),
        pltpu.VMEM((C, DW), jnp.int32),
        pltpu.VMEM((C, DW), jnp.int32),
        pltpu.VMEM((CN, DW), jnp.int32),
        pltpu.VMEM((CN, DW), jnp.int32),
        pltpu.VMEM((BPW,), jnp.float32),
        pltpu.VMEM((BPW,), jnp.float32),
        pltpu.VMEM((BPW * NNEG,), jnp.float32),
        pltpu.VMEM((BPW,), jnp.float32),
        pltpu.VMEM((BPW * NNEG,), jnp.float32),
        pltpu.SemaphoreType.DMA,
        pltpu.SemaphoreType.DMA,
        pltpu.SemaphoreType.DMA,
    ],
)
def _sc_volumes(pos_u_h, pos_w_h, negf_h, zzw_h, zzc_h, vw_h, vc_h,
                tv_h, pv_h, nv_h, pi_h, ni_h,
                idx_u, idx_w, idx_n, rows_u0, rows_u1,
                rows_w0, rows_w1, rows_n0, rows_n1,
                o_tv, o_pv, o_nv, o_pi, o_ni, sem0, sem1, semv):
    wid = lax.axis_index("c") * NS + lax.axis_index("s")
    base = wid * BPW

    pltpu.sync_copy(pos_u_h.at[pl.ds(base, BPW)], idx_u)
    pltpu.sync_copy(pos_w_h.at[pl.ds(base, BPW)], idx_w)
    pltpu.sync_copy(negf_h.at[pl.ds(base * NNEG, BPW * NNEG)], idx_n)

    for g4 in range(BPW // 128):
        pltpu.async_copy(vw_h.at[idx_u.at[pl.ds(g4 * 128, 128)]],
                         o_tv.at[pl.ds(g4 * 128, 128)], semv)
        pltpu.async_copy(vc_h.at[idx_w.at[pl.ds(g4 * 128, 128)]],
                         o_pv.at[pl.ds(g4 * 128, 128)], semv)

    def vol_issue(gi, _):
        voff = pl.multiple_of(gi * 128, 8)
        pltpu.async_copy(vc_h.at[idx_n.at[pl.ds(voff, 128)]],
                         o_nv.at[pl.ds(voff, 128)], semv)
        return 0

    lax.fori_loop(0, BPW * NNEG // 128, vol_issue, 0)

    sems = (sem0, sem1)
    rows_u = (rows_u0, rows_u1)
    rows_w = (rows_w0, rows_w1)
    rows_n = (rows_n0, rows_n1)
    zero16 = jnp.zeros((16,), jnp.float32)
    lanes = lax.iota(jnp.int32, 16)

    def issue(ci, sl):
        sem = sems[sl]
        off = pl.multiple_of(ci * C, 8)
        noff = pl.multiple_of(ci * CN, 8)
        pltpu.async_copy(zzw_h.at[idx_u.at[pl.ds(off, C)]],
                         rows_u[sl], sem)
        pltpu.async_copy(zzc_h.at[idx_w.at[pl.ds(off, C)]],
                         rows_w[sl], sem)
        pltpu.async_copy(zzc_h.at[idx_n.at[pl.ds(noff, 128)]],
                         rows_n[sl].at[pl.ds(0, 128)], sem)
        pltpu.async_copy(zzc_h.at[idx_n.at[pl.ds(noff + 128, 128)]],
                         rows_n[sl].at[pl.ds(128, 128)], sem)
        pltpu.async_copy(zzc_h.at[idx_n.at[pl.ds(noff + 256, CN - 256)]],
                         rows_n[sl].at[pl.ds(256, CN - 256)], sem)

    def drain(sl):
        sem = sems[sl]
        pltpu.make_async_copy(zzw_h.at[pl.ds(0, C)], rows_u[sl], sem).wait()
        pltpu.make_async_copy(zzc_h.at[pl.ds(0, C)], rows_w[sl], sem).wait()
        pltpu.make_async_copy(zzc_h.at[pl.ds(0, CN)], rows_n[sl], sem).wait()

    def compute(ci, sl):
        ru = rows_u[sl]
        rw = rows_w[sl]
        rn = rows_n[sl]

        def pos_d(d, pi_):
            dv = jnp.bitwise_and(d + lanes, DH - 1)
            uzw = plsc.load_gather(ru, [lanes, dv])
            uZw = plsc.load_gather(ru, [lanes, dv + DH])
            wzw = plsc.load_gather(rw, [lanes, dv])
            wZw = plsc.load_gather(rw, [lanes, dv + DH])
            lo, hi = _int_term(uzw, uZw, wzw, wZw)
            return pi_ + lo + hi

        pi_ = lax.fori_loop(0, DH, pos_d, zero16, unroll=4)
        eoff = pl.multiple_of(ci * C, 8)
        o_pi[pl.ds(eoff, 16)] = pi_

        def grp(g, _):
            rvec = lanes + g * 16
            evec = lax.div(rvec, jnp.int32(NNEG))

            def neg_d(d, ni):
                dv = jnp.bitwise_and(d + lanes, DH - 1)
                nzw = plsc.load_gather(rn, [rvec, dv])
                nZw = plsc.load_gather(rn, [rvec, dv + DH])
                uzw = plsc.load_gather(ru, [evec, dv])
                uZw = plsc.load_gather(ru, [evec, dv + DH])
                lo, hi = _int_term(nzw, nZw, uzw, uZw)
                return ni + lo + hi

            ni = lax.fori_loop(0, DH, neg_d, zero16, unroll=4)
            foff = pl.multiple_of(ci * CN + g * 16, 8)
            o_ni[pl.ds(foff, 16)] = ni
            return 0

        lax.fori_loop(0, NG, grp, 0)

    issue(0, 0)

    def outer(p, _):
        for b in range(2):
            ci = p * 2 + b

            @pl.when(ci + 1 < NCHUNK)
            def _():
                issue(ci + 1, 1 - b)

            drain(b)
            compute(ci, b)
        return 0

    lax.fori_loop(0, NCHUNK // 2, outer, 0)

    pltpu.make_async_copy(vw_h.at[pl.ds(0, BPW)], o_tv, semv).wait()
    pltpu.make_async_copy(vw_h.at[pl.ds(0, BPW)], o_pv, semv).wait()
    pltpu.make_async_copy(vc_h.at[pl.ds(0, BPW * NNEG)], o_nv, semv).wait()

    pltpu.sync_copy(o_tv, tv_h.at[pl.ds(base, BPW)])
    pltpu.sync_copy(o_pv, pv_h.at[pl.ds(base, BPW)])
    pltpu.sync_copy(o_nv, nv_h.at[pl.ds(base * NNEG, BPW * NNEG)])
    pltpu.sync_copy(o_pi, pi_h.at[pl.ds(base, BPW)])
    pltpu.sync_copy(o_ni, ni_h.at[pl.ds(base * NNEG, BPW * NNEG)])


def kernel(pos_u, pos_w, neg_w, W_word, W_ctx):
    zzw, zzc, vw, vc = _transform_tables(W_word, W_ctx)
    neg_flat = neg_w.reshape(-1)
    tv, pv, nvf, pi, nif = _sc_volumes(
        pos_u, pos_w, neg_flat, zzw, zzc, vw.reshape(V), vc.reshape(V))
    return (tv, pv, nvf.reshape(B, NNEG), pi, nif.reshape(B, NNEG))
